# Initial kernel scaffold; baseline (speedup 1.0000x reference)
import jax
import jax.numpy as jnp
from jax.experimental import pallas as pl


def kernel(x_l, x_n, edge_index_l2n, edge_index_n2l, params):
    raise NotImplementedError("write your pallas kernel here")


# speedup vs baseline: 8.2659x; 8.2659x over previous
import functools
import jax
import jax.numpy as jnp
from jax.experimental import pallas as pl

H = 4
D = 32
HD = H * D
SLOPE = 0.2
LN_EPS = 1e-5

ROW_BLK = 1000


def _mm_attn_body(x_ref, w_ref, a_ref, f_ref, e_ref):
    f = jnp.dot(x_ref[...], w_ref[...], preferred_element_type=jnp.float32)
    f_ref[...] = f
    e_ref[...] = jnp.dot(f, a_ref[...], preferred_element_type=jnp.float32)


def _mm_attn(x, w, ablk):
    n, k = x.shape
    m = w.shape[1]
    ha = ablk.shape[1]
    grid = (n // ROW_BLK,)
    return pl.pallas_call(
        _mm_attn_body,
        grid=grid,
        in_specs=[
            pl.BlockSpec((ROW_BLK, k), lambda i: (i, 0)),
            pl.BlockSpec((k, m), lambda i: (0, 0)),
            pl.BlockSpec((m, ha), lambda i: (0, 0)),
        ],
        out_specs=[
            pl.BlockSpec((ROW_BLK, m), lambda i: (i, 0)),
            pl.BlockSpec((ROW_BLK, ha), lambda i: (i, 0)),
        ],
        out_shape=[
            jax.ShapeDtypeStruct((n, m), jnp.float32),
            jax.ShapeDtypeStruct((n, ha), jnp.float32),
        ],
    )(x, w, ablk)


def _mm_body(x_ref, w_ref, o_ref):
    o_ref[...] = jnp.dot(x_ref[...], w_ref[...], preferred_element_type=jnp.float32)


def _matmul(x, w):
    n, k = x.shape
    m = w.shape[1]
    grid = (n // ROW_BLK,)
    return pl.pallas_call(
        _mm_body,
        grid=grid,
        in_specs=[
            pl.BlockSpec((ROW_BLK, k), lambda i: (i, 0)),
            pl.BlockSpec((k, m), lambda i: (0, 0)),
        ],
        out_specs=pl.BlockSpec((ROW_BLK, m), lambda i: (i, 0)),
        out_shape=jax.ShapeDtypeStruct((n, m), jnp.float32),
    )(x, w)


def _post_ln_elu_body(rst_ref, res_ref, bias_ref, g_ref, b_ref, o_ref):
    x = rst_ref[...] + res_ref[...] + bias_ref[...]
    mu = jnp.mean(x, axis=-1, keepdims=True)
    var = jnp.mean((x - mu) ** 2, axis=-1, keepdims=True)
    y = (x - mu) * jax.lax.rsqrt(var + LN_EPS) * g_ref[...] + b_ref[...]
    o_ref[...] = jnp.where(y > 0, y, jnp.exp(y) - 1.0)


def _post_ln_elu(rst, res, bias, g, b):
    n, m = rst.shape
    grid = (n // ROW_BLK,)
    bias2 = bias.reshape(1, m)
    g2 = g.reshape(1, m)
    b2 = b.reshape(1, m)
    return pl.pallas_call(
        _post_ln_elu_body,
        grid=grid,
        in_specs=[
            pl.BlockSpec((ROW_BLK, m), lambda i: (i, 0)),
            pl.BlockSpec((ROW_BLK, m), lambda i: (i, 0)),
            pl.BlockSpec((1, m), lambda i: (0, 0)),
            pl.BlockSpec((1, m), lambda i: (0, 0)),
            pl.BlockSpec((1, m), lambda i: (0, 0)),
        ],
        out_specs=pl.BlockSpec((ROW_BLK, m), lambda i: (i, 0)),
        out_shape=jax.ShapeDtypeStruct((n, m), jnp.float32),
    )(rst, res, bias2, g2, b2)


def _final_body(rst_ref, res_ref, bias_ref, hm_ref, g_ref, b_ref, o_ref, s_ref):
    x = rst_ref[...] + res_ref[...] + bias_ref[...]
    x = jnp.dot(x, hm_ref[...], preferred_element_type=jnp.float32)
    mu = jnp.mean(x, axis=-1, keepdims=True)
    var = jnp.mean((x - mu) ** 2, axis=-1, keepdims=True)
    y = (x - mu) * jax.lax.rsqrt(var + LN_EPS) * g_ref[...] + b_ref[...]
    y = jnp.where(y > 0, y, jnp.exp(y) - 1.0)
    o_ref[...] = y
    s = jnp.sum(y, axis=0, keepdims=True)
    s_ref[...] = jnp.broadcast_to(s, s_ref.shape)


def _final_ln_elu(rst, res, bias, g, b):
    n, m = rst.shape
    grid = (n // ROW_BLK,)
    nblk = n // ROW_BLK
    bias2 = bias.reshape(1, m)
    g2 = g.reshape(1, D)
    b2 = b.reshape(1, D)
    hm = jnp.zeros((m, D), jnp.float32)
    for h in range(H):
        hm = hm.at[h * D + jnp.arange(D), jnp.arange(D)].set(1.0 / H)
    y, partial = pl.pallas_call(
        _final_body,
        grid=grid,
        in_specs=[
            pl.BlockSpec((ROW_BLK, m), lambda i: (i, 0)),
            pl.BlockSpec((ROW_BLK, m), lambda i: (i, 0)),
            pl.BlockSpec((1, m), lambda i: (0, 0)),
            pl.BlockSpec((m, D), lambda i: (0, 0)),
            pl.BlockSpec((1, D), lambda i: (0, 0)),
            pl.BlockSpec((1, D), lambda i: (0, 0)),
        ],
        out_specs=[
            pl.BlockSpec((ROW_BLK, D), lambda i: (i, 0)),
            pl.BlockSpec((1, 8, D), lambda i: (i, 0, 0)),
        ],
        out_shape=[
            jax.ShapeDtypeStruct((n, D), jnp.float32),
            jax.ShapeDtypeStruct((nblk, 8, D), jnp.float32),
        ],
    )(rst, res, bias2, hm, g2, b2)
    mean = jnp.sum(partial[:, 0, :], axis=0, keepdims=True) / n
    return y, mean


def _attn_blk(attn):
    ablk = jnp.zeros((HD, H), jnp.float32)
    for h in range(H):
        ablk = ablk.at[h * D + jnp.arange(D), h].set(attn[h])
    return ablk


def _edge_softmax_agg(fs, el, er, src, dst, num_dst):
    e = el[src] + er[dst]
    e = jnp.where(e >= 0, e, SLOPE * e)
    m = jax.ops.segment_max(e, dst, num_segments=num_dst)
    m = jnp.where(jnp.isfinite(m), m, 0.0)
    ex = jnp.exp(e - m[dst])
    s = jax.ops.segment_sum(ex, dst, num_segments=num_dst)
    a = ex / (s[dst] + 1e-16)
    msg = fs[src].reshape(-1, H, D) * a[:, :, None]
    rst = jax.ops.segment_sum(msg.reshape(-1, HD), dst, num_segments=num_dst)
    return rst


def _gat_dense(p, h_src, h_dst):
    W_src = p['W'] if 'W' in p else p['W_src']
    W_dst = p['W'] if 'W' in p else p['W_dst']
    fs, el = _mm_attn(h_src, W_src, _attn_blk(p['attn_l']))
    fd, er = _mm_attn(h_dst, W_dst, _attn_blk(p['attn_r']))
    if 'W_res' in p:
        res = _matmul(h_dst, p['W_res'])
    else:
        res = h_dst
    return fs, el, er, res


def kernel(x_l, x_n, edge_index_l2n, edge_index_n2l, params):
    src1, dst1 = edge_index_l2n[0], edge_index_l2n[1]
    src2, dst2 = edge_index_n2l[0], edge_index_n2l[1]
    n_l = x_l.shape[0]
    n_n = x_n.shape[0]

    h_l, h_n = x_l, x_n
    for i in range(2):
        pc = params['conv'][i]
        fs, el, er, res_n = _gat_dense(pc['l2n'], h_l, h_n)
        rst_n = _edge_softmax_agg(fs, el, er, src1, dst1, n_n)
        fs2, el2, er2, res_l = _gat_dense(pc['n2l'], h_n, h_l)
        rst_l = _edge_softmax_agg(fs2, el2, er2, src2, dst2, n_l)
        gn, bn = params['norms'][i]['n']
        gl, bl = params['norms'][i]['l']
        h_n = _post_ln_elu(rst_n, res_n, pc['l2n']['bias'], gn, bn)
        h_l = _post_ln_elu(rst_l, res_l, pc['n2l']['bias'], gl, bl)

    pc = params['conv'][2]
    fs, el, er, res_n = _gat_dense(pc['l2n'], h_l, h_n)
    rst_n = _edge_softmax_agg(fs, el, er, src1, dst1, n_n)
    fs2, el2, er2, res_l = _gat_dense(pc['n2l'], h_n, h_l)
    rst_l = _edge_softmax_agg(fs2, el2, er2, src2, dst2, n_l)
    gn, bn = params['norms'][2]['n']
    gl, bl = params['norms'][2]['l']
    _, mean_n = _final_ln_elu(rst_n, res_n, pc['l2n']['bias'], gn, bn)
    _, mean_l = _final_ln_elu(rst_l, res_l, pc['n2l']['bias'], gl, bl)

    hg = mean_l + mean_n
    hg = jax.nn.relu(hg @ params['fc1_W'] + params['fc1_b'])
    return hg @ params['out_W'] + params['out_b']


# speedup vs baseline: 9.8045x; 1.1861x over previous
import functools
import jax
import jax.numpy as jnp
from jax.experimental import pallas as pl

H = 4
D = 32
HD = H * D
SLOPE = 0.2
LN_EPS = 1e-5

ROW_BLK = 1000


def _mm_attn_body(x_ref, w_ref, a_ref, f_ref, e_ref):
    f = jnp.dot(x_ref[...], w_ref[...], preferred_element_type=jnp.float32)
    f_ref[...] = f
    e_ref[...] = jnp.dot(f, a_ref[...], preferred_element_type=jnp.float32)


def _mm_attn(x, w, ablk):
    n, k = x.shape
    m = w.shape[1]
    ha = ablk.shape[1]
    grid = (n // ROW_BLK,)
    return pl.pallas_call(
        _mm_attn_body,
        grid=grid,
        in_specs=[
            pl.BlockSpec((ROW_BLK, k), lambda i: (i, 0)),
            pl.BlockSpec((k, m), lambda i: (0, 0)),
            pl.BlockSpec((m, ha), lambda i: (0, 0)),
        ],
        out_specs=[
            pl.BlockSpec((ROW_BLK, m), lambda i: (i, 0)),
            pl.BlockSpec((ROW_BLK, ha), lambda i: (i, 0)),
        ],
        out_shape=[
            jax.ShapeDtypeStruct((n, m), jnp.float32),
            jax.ShapeDtypeStruct((n, ha), jnp.float32),
        ],
    )(x, w, ablk)


def _mm_body(x_ref, w_ref, o_ref):
    o_ref[...] = jnp.dot(x_ref[...], w_ref[...], preferred_element_type=jnp.float32)


def _matmul(x, w):
    n, k = x.shape
    m = w.shape[1]
    grid = (n // ROW_BLK,)
    return pl.pallas_call(
        _mm_body,
        grid=grid,
        in_specs=[
            pl.BlockSpec((ROW_BLK, k), lambda i: (i, 0)),
            pl.BlockSpec((k, m), lambda i: (0, 0)),
        ],
        out_specs=pl.BlockSpec((ROW_BLK, m), lambda i: (i, 0)),
        out_shape=jax.ShapeDtypeStruct((n, m), jnp.float32),
    )(x, w)


def _rep_mat():
    rm = jnp.zeros((H, HD), jnp.float32)
    for h in range(H):
        rm = rm.at[h, h * D + jnp.arange(D)].set(1.0)
    return rm


def _post_ln_elu_body(rst_ref, s_ref, rm_ref, res_ref, bias_ref, g_ref, b_ref, o_ref):
    srep = jnp.dot(s_ref[...], rm_ref[...], preferred_element_type=jnp.float32)
    x = rst_ref[...] / (srep + 1e-16) + res_ref[...] + bias_ref[...]
    mu = jnp.mean(x, axis=-1, keepdims=True)
    var = jnp.mean((x - mu) ** 2, axis=-1, keepdims=True)
    y = (x - mu) * jax.lax.rsqrt(var + LN_EPS) * g_ref[...] + b_ref[...]
    o_ref[...] = jnp.where(y > 0, y, jnp.exp(y) - 1.0)


def _post_ln_elu(rst, s, res, bias, g, b):
    n, m = rst.shape
    grid = (n // ROW_BLK,)
    bias2 = bias.reshape(1, m)
    g2 = g.reshape(1, m)
    b2 = b.reshape(1, m)
    return pl.pallas_call(
        _post_ln_elu_body,
        grid=grid,
        in_specs=[
            pl.BlockSpec((ROW_BLK, m), lambda i: (i, 0)),
            pl.BlockSpec((ROW_BLK, H), lambda i: (i, 0)),
            pl.BlockSpec((H, m), lambda i: (0, 0)),
            pl.BlockSpec((ROW_BLK, m), lambda i: (i, 0)),
            pl.BlockSpec((1, m), lambda i: (0, 0)),
            pl.BlockSpec((1, m), lambda i: (0, 0)),
            pl.BlockSpec((1, m), lambda i: (0, 0)),
        ],
        out_specs=pl.BlockSpec((ROW_BLK, m), lambda i: (i, 0)),
        out_shape=jax.ShapeDtypeStruct((n, m), jnp.float32),
    )(rst, s, _rep_mat(), res, bias2, g2, b2)


def _final_body(rst_ref, sd_ref, rm_ref, res_ref, bias_ref, hm_ref, g_ref, b_ref, o_ref, s_ref):
    srep = jnp.dot(sd_ref[...], rm_ref[...], preferred_element_type=jnp.float32)
    x = rst_ref[...] / (srep + 1e-16) + res_ref[...] + bias_ref[...]
    x = jnp.dot(x, hm_ref[...], preferred_element_type=jnp.float32)
    mu = jnp.mean(x, axis=-1, keepdims=True)
    var = jnp.mean((x - mu) ** 2, axis=-1, keepdims=True)
    y = (x - mu) * jax.lax.rsqrt(var + LN_EPS) * g_ref[...] + b_ref[...]
    y = jnp.where(y > 0, y, jnp.exp(y) - 1.0)
    o_ref[...] = y
    s = jnp.sum(y, axis=0, keepdims=True)
    s_ref[...] = jnp.broadcast_to(s, s_ref.shape)


def _final_ln_elu(rst, s, res, bias, g, b):
    n, m = rst.shape
    grid = (n // ROW_BLK,)
    nblk = n // ROW_BLK
    bias2 = bias.reshape(1, m)
    g2 = g.reshape(1, D)
    b2 = b.reshape(1, D)
    hm = jnp.zeros((m, D), jnp.float32)
    for h in range(H):
        hm = hm.at[h * D + jnp.arange(D), jnp.arange(D)].set(1.0 / H)
    y, partial = pl.pallas_call(
        _final_body,
        grid=grid,
        in_specs=[
            pl.BlockSpec((ROW_BLK, m), lambda i: (i, 0)),
            pl.BlockSpec((ROW_BLK, H), lambda i: (i, 0)),
            pl.BlockSpec((H, m), lambda i: (0, 0)),
            pl.BlockSpec((ROW_BLK, m), lambda i: (i, 0)),
            pl.BlockSpec((1, m), lambda i: (0, 0)),
            pl.BlockSpec((m, D), lambda i: (0, 0)),
            pl.BlockSpec((1, D), lambda i: (0, 0)),
            pl.BlockSpec((1, D), lambda i: (0, 0)),
        ],
        out_specs=[
            pl.BlockSpec((ROW_BLK, D), lambda i: (i, 0)),
            pl.BlockSpec((1, 8, D), lambda i: (i, 0, 0)),
        ],
        out_shape=[
            jax.ShapeDtypeStruct((n, D), jnp.float32),
            jax.ShapeDtypeStruct((nblk, 8, D), jnp.float32),
        ],
    )(rst, s, _rep_mat(), res, bias2, hm, g2, b2)
    mean = jnp.sum(partial[:, 0, :], axis=0, keepdims=True) / n
    return y, mean


def _attn_blk(attn):
    ablk = jnp.zeros((HD, H), jnp.float32)
    for h in range(H):
        ablk = ablk.at[h * D + jnp.arange(D), h].set(attn[h])
    return ablk


def _edge_softmax_agg(fs, el, er, src, dst, num_dst):
    e = el[src] + er[dst]
    e = jnp.where(e >= 0, e, SLOPE * e)
    m = jax.ops.segment_max(e, dst, num_segments=num_dst)
    m = jnp.where(jnp.isfinite(m), m, 0.0)
    ex = jnp.exp(e - m[dst])
    w = (fs[src].reshape(-1, H, D) * ex[:, :, None]).reshape(-1, HD)
    cat = jnp.concatenate([w, ex], axis=1)
    out = jax.ops.segment_sum(cat, dst, num_segments=num_dst)
    return out[:, :HD], out[:, HD:]


def _gat_dense(p, h_src, h_dst):
    W_src = p['W'] if 'W' in p else p['W_src']
    W_dst = p['W'] if 'W' in p else p['W_dst']
    fs, el = _mm_attn(h_src, W_src, _attn_blk(p['attn_l']))
    fd, er = _mm_attn(h_dst, W_dst, _attn_blk(p['attn_r']))
    if 'W_res' in p:
        res = _matmul(h_dst, p['W_res'])
    else:
        res = h_dst
    return fs, el, er, res


def kernel(x_l, x_n, edge_index_l2n, edge_index_n2l, params):
    src1, dst1 = edge_index_l2n[0], edge_index_l2n[1]
    src2, dst2 = edge_index_n2l[0], edge_index_n2l[1]
    n_l = x_l.shape[0]
    n_n = x_n.shape[0]

    h_l, h_n = x_l, x_n
    for i in range(2):
        pc = params['conv'][i]
        fs, el, er, res_n = _gat_dense(pc['l2n'], h_l, h_n)
        rst_n, s_n = _edge_softmax_agg(fs, el, er, src1, dst1, n_n)
        fs2, el2, er2, res_l = _gat_dense(pc['n2l'], h_n, h_l)
        rst_l, s_l = _edge_softmax_agg(fs2, el2, er2, src2, dst2, n_l)
        gn, bn = params['norms'][i]['n']
        gl, bl = params['norms'][i]['l']
        h_n = _post_ln_elu(rst_n, s_n, res_n, pc['l2n']['bias'], gn, bn)
        h_l = _post_ln_elu(rst_l, s_l, res_l, pc['n2l']['bias'], gl, bl)

    pc = params['conv'][2]
    fs, el, er, res_n = _gat_dense(pc['l2n'], h_l, h_n)
    rst_n, s_n = _edge_softmax_agg(fs, el, er, src1, dst1, n_n)
    fs2, el2, er2, res_l = _gat_dense(pc['n2l'], h_n, h_l)
    rst_l, s_l = _edge_softmax_agg(fs2, el2, er2, src2, dst2, n_l)
    gn, bn = params['norms'][2]['n']
    gl, bl = params['norms'][2]['l']
    _, mean_n = _final_ln_elu(rst_n, s_n, res_n, pc['l2n']['bias'], gn, bn)
    _, mean_l = _final_ln_elu(rst_l, s_l, res_l, pc['n2l']['bias'], gl, bl)

    hg = mean_l + mean_n
    hg = jax.nn.relu(hg @ params['fc1_W'] + params['fc1_b'])
    return hg @ params['out_W'] + params['out_b']
